# Initial kernel scaffold; baseline (speedup 1.0000x reference)
import jax
import jax.numpy as jnp
from jax.experimental import pallas as pl


def kernel(x, xtype, weight, w_comp):
    raise NotImplementedError("write your pallas kernel here")


# speedup vs baseline: 1.5125x; 1.5125x over previous
import jax
import jax.numpy as jnp
from jax.experimental import pallas as pl

N = 160000
IN_DIM = 16
OUT_DIM = 16
NUM_TRANSFORMS = 100
NUM_BASES = 8

ROWS = 4000


def _tile_kernel(xt_ref, x_ref, w2_ref, wce_ref, o_ref):
    x = x_ref[...]
    z = jnp.dot(x, w2_ref[...], preferred_element_type=jnp.float32)
    idx = xt_ref[0]
    tids = jax.lax.broadcasted_iota(jnp.int32, (ROWS, NUM_TRANSFORMS), 1)
    onehot = (idx == tids).astype(jnp.float32)
    c_exp = jnp.dot(onehot, wce_ref[...],
                    preferred_element_type=jnp.float32)
    p = z * c_exp
    acc = p[:, 0:OUT_DIM]
    for b in range(1, NUM_BASES):
        acc = acc + p[:, b * OUT_DIM:(b + 1) * OUT_DIM]
    o_ref[...] = acc


@jax.jit
def _run(x, xtype3, w2, wce):
    grid = N // ROWS
    return pl.pallas_call(
        _tile_kernel,
        grid=(grid,),
        in_specs=[
            pl.BlockSpec((1, ROWS, 1), lambda i: (i, 0, 0)),
            pl.BlockSpec((ROWS, IN_DIM), lambda i: (i, 0)),
            pl.BlockSpec((IN_DIM, NUM_BASES * OUT_DIM), lambda i: (0, 0)),
            pl.BlockSpec((NUM_TRANSFORMS, NUM_BASES * OUT_DIM), lambda i: (0, 0)),
        ],
        out_specs=pl.BlockSpec((ROWS, OUT_DIM), lambda i: (i, 0)),
        out_shape=jax.ShapeDtypeStruct((N, OUT_DIM), jnp.float32),
    )(xtype3, x, w2, wce)


def kernel(x, xtype, weight, w_comp):
    w2 = weight.reshape(NUM_BASES, IN_DIM, OUT_DIM).transpose(1, 0, 2)
    w2 = w2.reshape(IN_DIM, NUM_BASES * OUT_DIM)
    wce = jnp.repeat(w_comp, OUT_DIM, axis=1)
    xtype3 = xtype.astype(jnp.int32).reshape(N // ROWS, ROWS, 1)
    return _run(x, xtype3, w2, wce)


# speedup vs baseline: 1.5476x; 1.0232x over previous
import functools

import jax
import jax.numpy as jnp
from jax import lax
from jax.experimental import pallas as pl
from jax.experimental.pallas import tpu as pltpu
from jax.experimental.pallas import tpu_sc as plsc

N = 160000
IN_DIM = 16
OUT_DIM = 16
NUM_TRANSFORMS = 100
NUM_BASES = 8
PAD_B = 16
ROWS = 4000

_NC = 2
_NS = 16
_NW = _NC * _NS
_BPW = N // _NW


_sc_mesh = plsc.VectorSubcoreMesh(core_axis_name="c", subcore_axis_name="s")


@functools.partial(
    pl.kernel,
    mesh=_sc_mesh,
    compiler_params=pltpu.CompilerParams(use_tc_tiling_on_sc=False),
    out_type=jax.ShapeDtypeStruct((N, PAD_B), jnp.float32),
    scratch_types=[
        pltpu.VMEM((_BPW,), jnp.int32),
        pltpu.VMEM((_BPW, PAD_B), jnp.float32),
        pltpu.SemaphoreType.DMA,
    ],
)
def _sc_gather(table_hbm, idx_hbm, out_hbm, idx_v, rows_v, sem):
    wid = lax.axis_index("s") * _NC + lax.axis_index("c")
    base = wid * _BPW
    pltpu.sync_copy(idx_hbm.at[pl.ds(base, _BPW)], idx_v)
    pltpu.async_copy(table_hbm.at[idx_v], rows_v, sem).wait()
    pltpu.sync_copy(rows_v, out_hbm.at[pl.ds(base, _BPW)])


def _tc_body(x_ref, c_ref, w2_ref, e_ref, o_ref):
    z = jnp.dot(x_ref[...], w2_ref[...],
                preferred_element_type=jnp.float32)
    ce = jnp.dot(c_ref[...], e_ref[...],
                 preferred_element_type=jnp.float32)
    p = z * ce
    acc = p[:, 0:OUT_DIM]
    for b in range(1, NUM_BASES):
        acc = acc + p[:, b * OUT_DIM:(b + 1) * OUT_DIM]
    o_ref[...] = acc


@jax.jit
def _run(x, xtype, w2, e, table16):
    c16 = _sc_gather(table16, xtype)
    grid = N // ROWS
    return pl.pallas_call(
        _tc_body,
        grid=(grid,),
        in_specs=[
            pl.BlockSpec((ROWS, IN_DIM), lambda i: (i, 0)),
            pl.BlockSpec((ROWS, PAD_B), lambda i: (i, 0)),
            pl.BlockSpec((IN_DIM, NUM_BASES * OUT_DIM), lambda i: (0, 0)),
            pl.BlockSpec((PAD_B, NUM_BASES * OUT_DIM), lambda i: (0, 0)),
        ],
        out_specs=pl.BlockSpec((ROWS, OUT_DIM), lambda i: (i, 0)),
        out_shape=jax.ShapeDtypeStruct((N, OUT_DIM), jnp.float32),
    )(x, c16, w2, e)


def kernel(x, xtype, weight, w_comp):
    w2 = weight.reshape(NUM_BASES, IN_DIM, OUT_DIM).transpose(1, 0, 2)
    w2 = w2.reshape(IN_DIM, NUM_BASES * OUT_DIM)
    table16 = jnp.pad(w_comp, ((0, 0), (0, PAD_B - NUM_BASES)))
    e = jnp.repeat(jnp.eye(PAD_B, NUM_BASES, dtype=jnp.float32), OUT_DIM,
                   axis=1)
    return _run(x, xtype.astype(jnp.int32), w2, e, table16)


# speedup vs baseline: 2.3740x; 1.5340x over previous
import functools

import jax
import jax.numpy as jnp
from jax import lax
from jax.experimental import pallas as pl
from jax.experimental.pallas import tpu as pltpu
from jax.experimental.pallas import tpu_sc as plsc

N = 160000
IN_DIM = 16
OUT_DIM = 16
NUM_TRANSFORMS = 100
NUM_BASES = 8
ROWS = 8000

_NC = 2
_NS = 16
_NW = _NC * _NS
_BPW = N // _NW
_GROUPS = _BPW // 16
_TAIL = _BPW - _GROUPS * 16


_sc_mesh = plsc.VectorSubcoreMesh(core_axis_name="c", subcore_axis_name="s")


@functools.partial(
    pl.kernel,
    mesh=_sc_mesh,
    compiler_params=pltpu.CompilerParams(use_tc_tiling_on_sc=False,
                                         needs_layout_passes=False),
    out_type=jax.ShapeDtypeStruct((N, NUM_BASES), jnp.float32),
    scratch_types=[
        pltpu.VMEM((_BPW + 16,), jnp.int32),
        pltpu.VMEM((NUM_TRANSFORMS, NUM_BASES), jnp.float32),
        pltpu.VMEM((_BPW + 16, NUM_BASES), jnp.float32),
        pltpu.SemaphoreType.DMA,
    ],
)
def _sc_gather(table_hbm, idx_hbm, out_hbm, idx_v, table_v, rows_v, sem):
    wid = lax.axis_index("s") * _NC + lax.axis_index("c")
    base = wid * _BPW
    pltpu.sync_copy(table_hbm, table_v)
    pltpu.sync_copy(idx_hbm.at[pl.ds(base, _BPW)], idx_v.at[pl.ds(0, _BPW)])
    lanes = lax.iota(jnp.int32, 16)
    maxid = jnp.full((16,), NUM_TRANSFORMS - 1, jnp.int32)

    def gather16(g):
        rows = g * 16 + lanes
        rowidx = jnp.clip(plsc.load_gather(idx_v, [rows]), 0, maxid)
        for j in range(NUM_BASES):
            col = jnp.full((16,), j, jnp.int32)
            vals = plsc.load_gather(table_v, [rowidx, col])
            plsc.store_scatter(rows_v, [rows, col], vals)

    def body(g, carry):
        gather16(g)
        return carry

    lax.fori_loop(0, _GROUPS + (1 if _TAIL else 0), body, 0)
    pltpu.sync_copy(rows_v.at[pl.ds(0, _BPW)], out_hbm.at[pl.ds(base, _BPW)])


def _tc_body(x_ref, c_ref, w2_ref, e_ref, s_ref, o_ref):
    z = jnp.dot(x_ref[...], w2_ref[...],
                preferred_element_type=jnp.float32)
    ce = jnp.dot(c_ref[...], e_ref[...],
                 preferred_element_type=jnp.float32)
    p = (z * ce).astype(jnp.bfloat16)
    o_ref[...] = jnp.dot(p, s_ref[...],
                         preferred_element_type=jnp.float32)


@jax.jit
def _run(x, xtype, w2, e, s, w_comp):
    c8 = _sc_gather(w_comp, xtype)
    grid = N // ROWS
    return pl.pallas_call(
        _tc_body,
        grid=(grid,),
        in_specs=[
            pl.BlockSpec((ROWS, IN_DIM), lambda i: (i, 0)),
            pl.BlockSpec((ROWS, NUM_BASES), lambda i: (i, 0)),
            pl.BlockSpec((IN_DIM, NUM_BASES * OUT_DIM), lambda i: (0, 0)),
            pl.BlockSpec((NUM_BASES, NUM_BASES * OUT_DIM), lambda i: (0, 0)),
            pl.BlockSpec((NUM_BASES * OUT_DIM, OUT_DIM), lambda i: (0, 0)),
        ],
        out_specs=pl.BlockSpec((ROWS, OUT_DIM), lambda i: (i, 0)),
        out_shape=jax.ShapeDtypeStruct((N, OUT_DIM), jnp.float32),
    )(x, c8, w2, e, s)


def kernel(x, xtype, weight, w_comp):
    w2 = weight.reshape(NUM_BASES, IN_DIM, OUT_DIM).transpose(1, 0, 2)
    w2 = w2.reshape(IN_DIM, NUM_BASES * OUT_DIM)
    e = jnp.repeat(jnp.eye(NUM_BASES, dtype=jnp.float32), OUT_DIM, axis=1)
    s = jnp.tile(jnp.eye(OUT_DIM, dtype=jnp.bfloat16), (NUM_BASES, 1))
    return _run(x, xtype.astype(jnp.int32), w2, e, s, w_comp)


# speedup vs baseline: 2.7171x; 1.1445x over previous
import functools

import jax
import jax.numpy as jnp
from jax import lax
from jax.experimental import pallas as pl
from jax.experimental.pallas import tpu as pltpu
from jax.experimental.pallas import tpu_sc as plsc

N = 160000
IN_DIM = 16
OUT_DIM = 16
NUM_TRANSFORMS = 100
NUM_BASES = 8
PAD_B = 16
ROWS = 8000
PACK = 8

_NC = 2
_NS = 16
_NW = _NC * _NS
_BPW = N // _NW
_GROUPS = (_BPW + 15) // 16
_FLAT = _BPW * PAD_B


_sc_mesh = plsc.VectorSubcoreMesh(core_axis_name="c", subcore_axis_name="s")


@functools.partial(
    pl.kernel,
    mesh=_sc_mesh,
    compiler_params=pltpu.CompilerParams(use_tc_tiling_on_sc=False,
                                         needs_layout_passes=False),
    out_type=jax.ShapeDtypeStruct((N * PAD_B,), jnp.float32),
    scratch_types=[
        pltpu.VMEM((_GROUPS * 16,), jnp.int32),
        pltpu.VMEM((NUM_TRANSFORMS * PAD_B,), jnp.float32),
        pltpu.VMEM((_GROUPS * 16 * PAD_B,), jnp.float32),
        pltpu.SemaphoreType.DMA,
    ],
)
def _sc_gather(table_hbm, idx_hbm, out_hbm, idx_v, table_v, rows_v, sem):
    wid = lax.axis_index("s") * _NC + lax.axis_index("c")
    base = wid * _BPW
    pltpu.sync_copy(table_hbm, table_v)
    pltpu.sync_copy(idx_hbm.at[pl.ds(base, _BPW)], idx_v.at[pl.ds(0, _BPW)])
    lanes = lax.iota(jnp.int32, 16)
    maxid = jnp.full((16,), NUM_TRANSFORMS - 1, jnp.int32)

    def body(g, carry):
        rows = g * 16 + lanes
        rowidx = jnp.clip(plsc.load_gather(idx_v, [rows]), 0, maxid)
        tbase = rowidx * PAD_B
        obase = rows * PAD_B
        for j in range(PAD_B):
            vals = plsc.load_gather(table_v, [tbase + j])
            plsc.store_scatter(rows_v, [obase + j], vals)
        return carry

    lax.fori_loop(0, _GROUPS, body, 0)
    pltpu.sync_copy(rows_v.at[pl.ds(0, _FLAT)],
                    out_hbm.at[pl.ds(wid * _FLAT, _FLAT)])


def _tc_body(x_ref, c_ref, m_ref, el_ref, o_ref):
    xb = x_ref[...].astype(jnp.bfloat16)
    cb = c_ref[...].astype(jnp.bfloat16)
    acc = None
    for b in range(NUM_BASES):
        zb = jnp.dot(xb, m_ref[b], preferred_element_type=jnp.float32)
        ceb = jnp.dot(cb, el_ref[b], preferred_element_type=jnp.float32)
        term = zb * ceb
        acc = term if acc is None else acc + term
    o_ref[...] = acc


@jax.jit
def _run(x2, xtype, m, el, table):
    cflat = _sc_gather(table, xtype)
    c2 = cflat.reshape(N // PACK, 128)
    grid = N // ROWS
    rb = ROWS // PACK
    out2 = pl.pallas_call(
        _tc_body,
        grid=(grid,),
        in_specs=[
            pl.BlockSpec((rb, 128), lambda i: (i, 0)),
            pl.BlockSpec((rb, 128), lambda i: (i, 0)),
            pl.BlockSpec((NUM_BASES, 128, 128), lambda i: (0, 0, 0)),
            pl.BlockSpec((NUM_BASES, 128, 128), lambda i: (0, 0, 0)),
        ],
        out_specs=pl.BlockSpec((rb, 128), lambda i: (i, 0)),
        out_shape=jax.ShapeDtypeStruct((N // PACK, 128), jnp.float32),
    )(x2, c2, m, el)
    return out2


def kernel(x, xtype, weight, w_comp):
    basis = weight.reshape(NUM_BASES, IN_DIM, OUT_DIM)
    eye8 = jnp.eye(PACK, dtype=jnp.float32)
    m = jnp.einsum("rs,bko->brkso", eye8, basis).reshape(
        NUM_BASES, 128, 128).astype(jnp.bfloat16)
    sel = jnp.zeros((NUM_BASES, PAD_B, OUT_DIM), jnp.float32)
    sel = sel.at[jnp.arange(NUM_BASES), jnp.arange(NUM_BASES), :].set(1.0)
    el = jnp.einsum("rs,bco->brcso", eye8, sel).reshape(
        NUM_BASES, 128, 128).astype(jnp.bfloat16)
    table = jnp.pad(w_comp, ((0, 0), (0, PAD_B - NUM_BASES))).reshape(-1)
    x2 = x.reshape(N // PACK, 128)
    out2 = _run(x2, xtype.astype(jnp.int32), m, el, table)
    return out2.reshape(N, OUT_DIM)


# speedup vs baseline: 2.7335x; 1.0060x over previous
import functools

import jax
import jax.numpy as jnp
from jax import lax
from jax.experimental import pallas as pl
from jax.experimental.pallas import tpu as pltpu
from jax.experimental.pallas import tpu_sc as plsc

N = 160000
IN_DIM = 16
OUT_DIM = 16
NUM_TRANSFORMS = 100
NUM_BASES = 8
PAD_B = 16
ROWS = 8000
PACK = 8

_NC = 2
_NS = 16
_NW = _NC * _NS
_BPW = N // _NW
_GROUPS = (_BPW + 15) // 16
_FLAT = _BPW * PAD_B


_sc_mesh = plsc.VectorSubcoreMesh(core_axis_name="c", subcore_axis_name="s")


@functools.partial(
    pl.kernel,
    mesh=_sc_mesh,
    compiler_params=pltpu.CompilerParams(use_tc_tiling_on_sc=False,
                                         needs_layout_passes=False),
    out_type=jax.ShapeDtypeStruct((N * PAD_B,), jnp.float32),
    scratch_types=[
        pltpu.VMEM((_GROUPS * 16,), jnp.int32),
        pltpu.VMEM((NUM_TRANSFORMS * PAD_B,), jnp.float32),
        pltpu.VMEM((_GROUPS * 16 * PAD_B,), jnp.float32),
        pltpu.SemaphoreType.DMA,
    ],
)
def _sc_gather(table_hbm, idx_hbm, out_hbm, idx_v, table_v, rows_v, sem):
    wid = lax.axis_index("s") * _NC + lax.axis_index("c")
    base = wid * _BPW
    pltpu.sync_copy(table_hbm, table_v)
    pltpu.sync_copy(idx_hbm.at[pl.ds(base, _BPW)], idx_v.at[pl.ds(0, _BPW)])
    lanes = lax.iota(jnp.int32, 16)
    maxid = jnp.full((16,), NUM_TRANSFORMS - 1, jnp.int32)
    zeros = jnp.zeros((16,), jnp.float32)

    def body(g, carry):
        rows = g * 16 + lanes
        rowidx = jnp.clip(plsc.load_gather(idx_v, [rows]), 0, maxid)
        tbase = rowidx * PAD_B
        obase = rows * PAD_B
        for j in range(NUM_BASES):
            vals = plsc.load_gather(table_v, [tbase + j])
            plsc.store_scatter(rows_v, [obase + j], vals)
        for j in range(NUM_BASES, PAD_B):
            plsc.store_scatter(rows_v, [obase + j], zeros)
        return carry

    lax.fori_loop(0, _GROUPS, body, 0)
    pltpu.sync_copy(rows_v.at[pl.ds(0, _FLAT)],
                    out_hbm.at[pl.ds(wid * _FLAT, _FLAT)])


def _tc_body(x_ref, c_ref, m_ref, el_ref, o_ref):
    xb = x_ref[...].astype(jnp.bfloat16)
    cb = c_ref[...].astype(jnp.bfloat16)
    acc = None
    for b in range(NUM_BASES):
        zb = jnp.dot(xb, m_ref[b], preferred_element_type=jnp.float32)
        ceb = jnp.dot(cb, el_ref[b], preferred_element_type=jnp.float32)
        term = zb * ceb
        acc = term if acc is None else acc + term
    o_ref[...] = acc


@jax.jit
def _run(x2, xtype, m, el, table):
    cflat = _sc_gather(table, xtype)
    c2 = cflat.reshape(N // PACK, 128)
    grid = N // ROWS
    rb = ROWS // PACK
    out2 = pl.pallas_call(
        _tc_body,
        grid=(grid,),
        in_specs=[
            pl.BlockSpec((rb, 128), lambda i: (i, 0)),
            pl.BlockSpec((rb, 128), lambda i: (i, 0)),
            pl.BlockSpec((NUM_BASES, 128, 128), lambda i: (0, 0, 0)),
            pl.BlockSpec((NUM_BASES, 128, 128), lambda i: (0, 0, 0)),
        ],
        out_specs=pl.BlockSpec((rb, 128), lambda i: (i, 0)),
        out_shape=jax.ShapeDtypeStruct((N // PACK, 128), jnp.float32),
    )(x2, c2, m, el)
    return out2


def kernel(x, xtype, weight, w_comp):
    basis = weight.reshape(NUM_BASES, IN_DIM, OUT_DIM)
    eye8 = jnp.eye(PACK, dtype=jnp.float32)
    m = jnp.einsum("rs,bko->brkso", eye8, basis).reshape(
        NUM_BASES, 128, 128).astype(jnp.bfloat16)
    sel = jnp.zeros((NUM_BASES, PAD_B, OUT_DIM), jnp.float32)
    sel = sel.at[jnp.arange(NUM_BASES), jnp.arange(NUM_BASES), :].set(1.0)
    el = jnp.einsum("rs,bco->brcso", eye8, sel).reshape(
        NUM_BASES, 128, 128).astype(jnp.bfloat16)
    table = jnp.pad(w_comp, ((0, 0), (0, PAD_B - NUM_BASES))).reshape(-1)
    x2 = x.reshape(N // PACK, 128)
    out2 = _run(x2, xtype.astype(jnp.int32), m, el, table)
    return out2.reshape(N, OUT_DIM)


# speedup vs baseline: 2.7354x; 1.0007x over previous
import functools

import jax
import jax.numpy as jnp
from jax import lax
from jax.experimental import pallas as pl
from jax.experimental.pallas import tpu as pltpu
from jax.experimental.pallas import tpu_sc as plsc

N = 160000
IN_DIM = 16
OUT_DIM = 16
NUM_TRANSFORMS = 100
NUM_BASES = 8
PAD_B = 16
ROWS = 8000
PACK = 8

_NC = 2
_NS = 16
_NW = _NC * _NS
_BPW = N // _NW
_GROUPS = (_BPW + 15) // 16
_FLAT = _BPW * PAD_B


_sc_mesh = plsc.VectorSubcoreMesh(core_axis_name="c", subcore_axis_name="s")


@functools.partial(
    pl.kernel,
    mesh=_sc_mesh,
    compiler_params=pltpu.CompilerParams(use_tc_tiling_on_sc=False,
                                         needs_layout_passes=False),
    out_type=jax.ShapeDtypeStruct((N * PAD_B,), jnp.float32),
    scratch_types=[
        pltpu.VMEM((_GROUPS * 16,), jnp.int32),
        pltpu.VMEM((NUM_TRANSFORMS * NUM_BASES,), jnp.float32),
        pltpu.VMEM((_GROUPS * 16 * PAD_B,), jnp.float32),
        pltpu.SemaphoreType.DMA,
    ],
)
def _sc_gather(table_hbm, idx_hbm, out_hbm, idx_v, table_v, rows_v, sem):
    wid = lax.axis_index("s") * _NC + lax.axis_index("c")
    base = wid * _BPW
    pltpu.sync_copy(table_hbm, table_v)
    pltpu.sync_copy(idx_hbm.at[pl.ds(base, _BPW)], idx_v.at[pl.ds(0, _BPW)])
    lanes = lax.iota(jnp.int32, 16)
    maxid = jnp.full((16,), NUM_TRANSFORMS - 1, jnp.int32)

    def body(g, carry):
        rows = g * 16 + lanes
        rowidx = jnp.clip(idx_v[pl.ds(g * 16, 16)], 0, maxid)
        tbase = rowidx * NUM_BASES
        obase = rows * PAD_B
        for j in range(NUM_BASES):
            vals = plsc.load_gather(table_v, [tbase + j])
            plsc.store_scatter(rows_v, [obase + j], vals)
        return carry

    lax.fori_loop(0, _GROUPS, body, 0)
    pltpu.sync_copy(rows_v.at[pl.ds(0, _FLAT)],
                    out_hbm.at[pl.ds(wid * _FLAT, _FLAT)])


def _tc_body(x_ref, c_ref, m_ref, el_ref, o_ref):
    rb = ROWS // PACK
    xb = x_ref[...].astype(jnp.bfloat16)
    col = lax.broadcasted_iota(jnp.int32, (rb, 128), 1)
    keep = (col % PAD_B) < NUM_BASES
    cb = jnp.where(keep, c_ref[...], 0.0).astype(jnp.bfloat16)
    acc = None
    for b in range(NUM_BASES):
        zb = jnp.dot(xb, m_ref[b], preferred_element_type=jnp.float32)
        ceb = jnp.dot(cb, el_ref[b], preferred_element_type=jnp.float32)
        term = zb * ceb
        acc = term if acc is None else acc + term
    o_ref[...] = acc


@jax.jit
def _run(x2, xtype, m, el, table):
    cflat = _sc_gather(table, xtype)
    c2 = cflat.reshape(N // PACK, 128)
    grid = N // ROWS
    rb = ROWS // PACK
    out2 = pl.pallas_call(
        _tc_body,
        grid=(grid,),
        in_specs=[
            pl.BlockSpec((rb, 128), lambda i: (i, 0)),
            pl.BlockSpec((rb, 128), lambda i: (i, 0)),
            pl.BlockSpec((NUM_BASES, 128, 128), lambda i: (0, 0, 0)),
            pl.BlockSpec((NUM_BASES, 128, 128), lambda i: (0, 0, 0)),
        ],
        out_specs=pl.BlockSpec((rb, 128), lambda i: (i, 0)),
        out_shape=jax.ShapeDtypeStruct((N // PACK, 128), jnp.float32),
    )(x2, c2, m, el)
    return out2


def kernel(x, xtype, weight, w_comp):
    basis = weight.reshape(NUM_BASES, IN_DIM, OUT_DIM)
    eye8 = jnp.eye(PACK, dtype=jnp.float32)
    m = jnp.einsum("rs,bko->brkso", eye8, basis).reshape(
        NUM_BASES, 128, 128).astype(jnp.bfloat16)
    sel = jnp.zeros((NUM_BASES, PAD_B, OUT_DIM), jnp.float32)
    sel = sel.at[jnp.arange(NUM_BASES), jnp.arange(NUM_BASES), :].set(1.0)
    el = jnp.einsum("rs,bco->brcso", eye8, sel).reshape(
        NUM_BASES, 128, 128).astype(jnp.bfloat16)
    table = w_comp.reshape(-1)
    x2 = x.reshape(N // PACK, 128)
    out2 = _run(x2, xtype.astype(jnp.int32), m, el, table)
    return out2.reshape(N, OUT_DIM)
